# baseline (speedup 1.0000x reference)
import jax
import jax.numpy as jnp
from jax.experimental import pallas as pl
from jax.experimental.pallas import tpu as pltpu

EMB = 384
NEXP = 8
HID = 1536
NCLS = 1000
TOPK = 2
_EPS = 1e-5


def _layernorm(v, g, b):
    mu = jnp.mean(v, axis=-1, keepdims=True)
    var = jnp.mean((v - mu) ** 2, axis=-1, keepdims=True)
    return (v - mu) / jnp.sqrt(var + _EPS) * g + b


def _mm_t(a, w):
    return jax.lax.dot_general(
        a, w, (((1,), (1,)), ((), ())), preferred_element_type=jnp.float32
    )


def _gelu(h):
    return 0.5 * h * (1.0 + jax.lax.erf(h * (1.0 / jnp.sqrt(2.0).astype(jnp.float32))))


def _body(cls_ref, pos_ref, rw_ref, rb_ref, g1_ref, c1_ref, bv_ref, bo_ref,
          g2_ref, c2_ref, b1_ref, b2_ref, ng_ref, nb_ref, hw_ref, hb_ref,
          wv_hbm, wo_hbm, w1_hbm, w2_hbm, out_ref,
          lv_ref, ls_ref, wv_s, wo_s, w1_s, w2_s, sems):
    tokrow = cls_ref[...] + pos_ref[...]
    logits = _mm_t(tokrow, rw_ref[...]) + rb_ref[...]
    lv_ref[:, 0:NEXP] = logits
    cp = pltpu.make_async_copy(lv_ref, ls_ref, sems.at[0])
    cp.start()
    cp.wait()

    m1 = ls_ref[0, 0]
    i1 = jnp.int32(0)
    for e in range(1, NEXP):
        v = ls_ref[0, e]
        better = v > m1
        i1 = jnp.where(better, jnp.int32(e), i1)
        m1 = jnp.where(better, v, m1)
    m2 = jnp.float32(-3.0e38)
    i2 = jnp.int32(0)
    for e in range(NEXP):
        v = ls_ref[0, e]
        better = jnp.logical_and(v > m2, jnp.int32(e) != i1)
        i2 = jnp.where(better, jnp.int32(e), i2)
        m2 = jnp.where(better, v, m2)

    cps = []
    for k, e in enumerate((i1, i2)):
        cps.append(pltpu.make_async_copy(wv_hbm.at[e], wv_s.at[k], sems.at[len(cps) + 1]))
        cps.append(pltpu.make_async_copy(wo_hbm.at[e], wo_s.at[k], sems.at[len(cps) + 1]))
        for j in range(4):
            sl = pl.ds(j * (HID // 4), HID // 4)
            cps.append(pltpu.make_async_copy(
                w1_hbm.at[e, sl, :], w1_s.at[k, sl, :], sems.at[len(cps) + 1]))
        for j in range(4):
            sl = pl.ds(j * (EMB // 4), EMB // 4)
            cps.append(pltpu.make_async_copy(
                w2_hbm.at[e, sl, :], w2_s.at[k, sl, :], sems.at[len(cps) + 1]))
    for c in cps:
        c.start()
    for c in cps:
        c.wait()

    def expert_out(e, wv, wo, w1, w2):
        xn = _layernorm(tokrow, g1_ref[e], c1_ref[e])
        v = _mm_t(xn, wv) + bv_ref[e]
        attn = _mm_t(v, wo) + bo_ref[e]
        hmid = tokrow + attn
        hn = _layernorm(hmid, g2_ref[e], c2_ref[e])
        h1 = _gelu(_mm_t(hn, w1) + b1_ref[e])
        m = _mm_t(h1, w2) + b2_ref[e]
        return hmid + m

    y1 = expert_out(i1, wv_s[0], wo_s[0], w1_s[0], w2_s[0])
    y2 = expert_out(i2, wv_s[1], wo_s[1], w1_s[1], w2_s[1])
    s = (y1 + y2) * (1.0 / TOPK)
    o = _layernorm(s, ng_ref[...], nb_ref[...])
    head = _mm_t(o, hw_ref[...]) + hb_ref[...]
    out_ref[...] = jnp.broadcast_to(head, out_ref.shape)


def kernel(x, patch_W, patch_b, cls_token, pos_embed, router_W, router_b,
           ln1_g, ln1_b, Wv, bv, Wo, bo, ln2_g, ln2_b, W1, b1, W2, b2,
           norm_g, norm_b, head_W, head_b):
    Bsz = x.shape[0]
    cls2 = cls_token.reshape(1, EMB)
    pos0 = pos_embed[:, 0, :].reshape(1, EMB)

    vmem = pl.BlockSpec(memory_space=pltpu.VMEM)
    hbm = pl.BlockSpec(memory_space=pl.ANY)

    out = pl.pallas_call(
        _body,
        in_specs=[vmem] * 16 + [hbm] * 4,
        out_specs=vmem,
        out_shape=jax.ShapeDtypeStruct((Bsz, NCLS), jnp.float32),
        scratch_shapes=[
            pltpu.VMEM((1, 128), jnp.float32),
            pltpu.SMEM((1, 128), jnp.float32),
            pltpu.VMEM((TOPK, EMB, EMB), jnp.float32),
            pltpu.VMEM((TOPK, EMB, EMB), jnp.float32),
            pltpu.VMEM((TOPK, HID, EMB), jnp.float32),
            pltpu.VMEM((TOPK, EMB, HID), jnp.float32),
            pltpu.SemaphoreType.DMA((21,)),
        ],
    )(cls2, pos0, router_W, router_b.reshape(1, NEXP),
      ln1_g.reshape(NEXP, 1, EMB), ln1_b.reshape(NEXP, 1, EMB),
      bv.reshape(NEXP, 1, EMB), bo.reshape(NEXP, 1, EMB),
      ln2_g.reshape(NEXP, 1, EMB), ln2_b.reshape(NEXP, 1, EMB),
      b1.reshape(NEXP, 1, HID), b2.reshape(NEXP, 1, EMB),
      norm_g.reshape(1, EMB), norm_b.reshape(1, EMB),
      head_W, head_b.reshape(1, NCLS),
      Wv, Wo, W1, W2)
    return out


# speedup vs baseline: 1.1463x; 1.1463x over previous
import jax
import jax.numpy as jnp
from jax.experimental import pallas as pl
from jax.experimental.pallas import tpu as pltpu

EMB = 384
NEXP = 8
HID = 1536
NCLS = 1000
TOPK = 2
_EPS = 1e-5


def _layernorm(v, g, b):
    mu = jnp.mean(v, axis=-1, keepdims=True)
    var = jnp.mean((v - mu) ** 2, axis=-1, keepdims=True)
    return (v - mu) / jnp.sqrt(var + _EPS) * g + b


def _mm_t(a, w):
    return jax.lax.dot_general(
        a, w, (((1,), (1,)), ((), ())), preferred_element_type=jnp.float32
    )


def _gelu(h):
    return 0.5 * h * (1.0 + jax.lax.erf(h * (1.0 / jnp.sqrt(2.0).astype(jnp.float32))))


def _router_body(s4_ref, w_ref, b_ref, idx_ref):
    tokrow = s4_ref[0:1, :] + s4_ref[1:2, :]
    logits = _mm_t(tokrow, w_ref[...]) + b_ref[...]
    lane = jax.lax.broadcasted_iota(jnp.int32, logits.shape, 1)
    m1 = jnp.max(logits, axis=-1, keepdims=True)
    i1 = jnp.min(jnp.where(logits == m1, lane, NEXP), axis=-1, keepdims=True)
    masked = jnp.where(lane == i1, jnp.full_like(logits, -3.0e38), logits)
    m2 = jnp.max(masked, axis=-1, keepdims=True)
    i2 = jnp.min(jnp.where(masked == m2, lane, NEXP), axis=-1, keepdims=True)
    idx_ref[...] = jnp.concatenate([i1, i2], axis=-1)


def _moe_body(eidx_ref, s4_ref,
              va_ref, b1a_ref, wva_ref, woa_ref, w1a_ref, w2a_ref,
              vb_ref, b1b_ref, wvb_ref, wob_ref, w1b_ref, w2b_ref,
              hw_ref, hb_ref, out_ref):
    tokrow = s4_ref[0:1, :] + s4_ref[1:2, :]

    def expert_out(v_ref, b1_ref, wv_ref, wo_ref, w1_ref, w2_ref):
        vv = v_ref[0]
        xn = _layernorm(tokrow, vv[0:1], vv[1:2])
        v = _mm_t(xn, wv_ref[0]) + vv[2:3]
        attn = _mm_t(v, wo_ref[0]) + vv[3:4]
        hmid = tokrow + attn
        hn = _layernorm(hmid, vv[4:5], vv[5:6])
        h1 = _gelu(_mm_t(hn, w1_ref[0]) + b1_ref[0])
        m = _mm_t(h1, w2_ref[0]) + vv[6:7]
        return hmid + m

    y1 = expert_out(va_ref, b1a_ref, wva_ref, woa_ref, w1a_ref, w2a_ref)
    y2 = expert_out(vb_ref, b1b_ref, wvb_ref, wob_ref, w1b_ref, w2b_ref)
    s = (y1 + y2) * (1.0 / TOPK)
    o = _layernorm(s, s4_ref[2:3, :], s4_ref[3:4, :])
    head = _mm_t(o, hw_ref[...]) + hb_ref[...]
    out_ref[...] = jnp.broadcast_to(head, out_ref.shape)


def kernel(x, patch_W, patch_b, cls_token, pos_embed, router_W, router_b,
           ln1_g, ln1_b, Wv, bv, Wo, bo, ln2_g, ln2_b, W1, b1, W2, b2,
           norm_g, norm_b, head_W, head_b):
    Bsz = x.shape[0]
    small4 = jnp.concatenate(
        [cls_token.reshape(1, EMB), pos_embed[:, 0, :].reshape(1, EMB),
         norm_g.reshape(1, EMB), norm_b.reshape(1, EMB)], axis=0)
    vecs7 = jnp.stack([ln1_g, ln1_b, bv, bo, ln2_g, ln2_b, b2], axis=1)
    b1r = b1.reshape(NEXP, 1, HID)

    idx = pl.pallas_call(
        _router_body,
        out_shape=jax.ShapeDtypeStruct((1, TOPK), jnp.int32),
    )(small4, router_W, router_b.reshape(1, NEXP))
    eidx = idx.reshape(TOPK)

    def _blk(shape, k):
        return pl.BlockSpec((1,) + shape, lambda i, e, _k=k: (e[_k], 0, 0))

    def _full(shape):
        nd = len(shape)
        return pl.BlockSpec(shape, lambda i, e: (0,) * nd)

    def _slot(k):
        return [
            _blk((7, EMB), k),
            _blk((1, HID), k),
            _blk((EMB, EMB), k),
            _blk((EMB, EMB), k),
            _blk((HID, EMB), k),
            _blk((EMB, HID), k),
        ]

    grid_spec = pltpu.PrefetchScalarGridSpec(
        num_scalar_prefetch=1,
        grid=(1,),
        in_specs=[_full((4, EMB))] + _slot(0) + _slot(1)
        + [_full((NCLS, EMB)), _full((1, NCLS))],
        out_specs=pl.BlockSpec((Bsz, NCLS), lambda i, e: (0, 0)),
    )

    expert_args = (vecs7, b1r, Wv, Wo, W1, W2)
    out = pl.pallas_call(
        _moe_body,
        grid_spec=grid_spec,
        out_shape=jax.ShapeDtypeStruct((Bsz, NCLS), jnp.float32),
    )(eidx, small4, *expert_args, *expert_args, head_W, head_b.reshape(1, NCLS))
    return out


# speedup vs baseline: 1.8772x; 1.6376x over previous
import jax
import jax.numpy as jnp
from jax.experimental import pallas as pl
from jax.experimental.pallas import tpu as pltpu

EMB = 384
NEXP = 8
HID = 1536
NCLS = 1000
TOPK = 2
_EPS = 1e-5


def _layernorm(v, g, b):
    mu = jnp.mean(v, axis=-1, keepdims=True)
    var = jnp.mean((v - mu) ** 2, axis=-1, keepdims=True)
    return (v - mu) / jnp.sqrt(var + _EPS) * g + b


def _mm_t(a, w):
    return jax.lax.dot_general(
        a, w, (((1,), (1,)), ((), ())), preferred_element_type=jnp.float32
    )


def _mm(a, w):
    return jax.lax.dot_general(
        a, w, (((1,), (0,)), ((), ())), preferred_element_type=jnp.float32
    )


def _gelu(h):
    return 0.5 * h * (1.0 + jax.lax.erf(h * (1.0 / jnp.sqrt(2.0).astype(jnp.float32))))


def _body(cls_ref, pos_ref, rw_ref, rb_ref,
          g1_ref, c1_ref, bv_ref, bo_ref, g2_ref, c2_ref, b1_ref, b2_ref,
          ng_ref, nb_ref, hw_ref, hb_ref,
          wv_hbm, wo_hbm, w1_hbm, w2_hbm, out_ref,
          iv_ref, is_ref, wv_s, wo_s, w1_s, w2_s, sems):
    tokrow = cls_ref[...] + pos_ref[...]
    logits = _mm_t(tokrow, rw_ref[...]) + rb_ref[...]

    lane = jax.lax.broadcasted_iota(jnp.int32, logits.shape, 1)
    m1 = jnp.max(logits, axis=-1, keepdims=True)
    i1 = jnp.min(jnp.where(logits == m1, lane, NEXP), axis=-1, keepdims=True)
    masked = jnp.where(lane == i1, jnp.full_like(logits, -3.0e38), logits)
    m2 = jnp.max(masked, axis=-1, keepdims=True)
    i2 = jnp.min(jnp.where(masked == m2, lane, NEXP), axis=-1, keepdims=True)

    iv_ref[:, 0:1] = i1
    iv_ref[:, 1:2] = i2
    cp = pltpu.make_async_copy(iv_ref, is_ref, sems.at[8])
    cp.start()

    oh1 = (lane == i1).astype(jnp.float32)
    oh2 = (lane == i2).astype(jnp.float32)

    cp.wait()
    e1 = is_ref[0, 0]
    e2 = is_ref[0, 1]

    cps = []
    for k, e in enumerate((e1, e2)):
        cps.append(pltpu.make_async_copy(wv_hbm.at[e], wv_s.at[k], sems.at[4 * k + 0]))
        cps.append(pltpu.make_async_copy(wo_hbm.at[e], wo_s.at[k], sems.at[4 * k + 1]))
        cps.append(pltpu.make_async_copy(w1_hbm.at[e], w1_s.at[k], sems.at[4 * k + 2]))
        cps.append(pltpu.make_async_copy(w2_hbm.at[e], w2_s.at[k], sems.at[4 * k + 3]))
    for c in cps:
        c.start()

    def pick(oh, ref):
        return _mm(oh, ref[...])

    def expert_out(k, oh):
        xn = _layernorm(tokrow, pick(oh, g1_ref), pick(oh, c1_ref))
        cps[4 * k + 0].wait()
        v = _mm_t(xn, wv_s[k]) + pick(oh, bv_ref)
        cps[4 * k + 1].wait()
        attn = _mm_t(v, wo_s[k]) + pick(oh, bo_ref)
        hmid = tokrow + attn
        hn = _layernorm(hmid, pick(oh, g2_ref), pick(oh, c2_ref))
        cps[4 * k + 2].wait()
        h1 = _gelu(_mm_t(hn, w1_s[k]) + pick(oh, b1_ref))
        cps[4 * k + 3].wait()
        m = _mm_t(h1, w2_s[k]) + pick(oh, b2_ref)
        return hmid + m

    y1 = expert_out(0, oh1)
    y2 = expert_out(1, oh2)
    s = (y1 + y2) * (1.0 / TOPK)
    o = _layernorm(s, ng_ref[...], nb_ref[...])
    head = _mm_t(o, hw_ref[...]) + hb_ref[...]
    out_ref[...] = jnp.broadcast_to(head, out_ref.shape)


def kernel(x, patch_W, patch_b, cls_token, pos_embed, router_W, router_b,
           ln1_g, ln1_b, Wv, bv, Wo, bo, ln2_g, ln2_b, W1, b1, W2, b2,
           norm_g, norm_b, head_W, head_b):
    Bsz = x.shape[0]
    vmem = pl.BlockSpec(memory_space=pltpu.VMEM)
    hbm = pl.BlockSpec(memory_space=pl.ANY)

    out = pl.pallas_call(
        _body,
        in_specs=[vmem] * 16 + [hbm] * 4,
        out_specs=vmem,
        out_shape=jax.ShapeDtypeStruct((Bsz, NCLS), jnp.float32),
        scratch_shapes=[
            pltpu.VMEM((1, 128), jnp.int32),
            pltpu.SMEM((1, 128), jnp.int32),
            pltpu.VMEM((TOPK, EMB, EMB), jnp.float32),
            pltpu.VMEM((TOPK, EMB, EMB), jnp.float32),
            pltpu.VMEM((TOPK, HID, EMB), jnp.float32),
            pltpu.VMEM((TOPK, EMB, HID), jnp.float32),
            pltpu.SemaphoreType.DMA((9,)),
        ],
    )(cls_token.reshape(1, EMB), pos_embed[:, 0, :].reshape(1, EMB),
      router_W, router_b.reshape(1, NEXP),
      ln1_g, ln1_b, bv, bo, ln2_g, ln2_b, b1, b2,
      norm_g.reshape(1, EMB), norm_b.reshape(1, EMB),
      head_W, head_b.reshape(1, NCLS),
      Wv, Wo, W1, W2)
    return out


# speedup vs baseline: 1.9441x; 1.0357x over previous
import jax
import jax.numpy as jnp
from jax.experimental import pallas as pl
from jax.experimental.pallas import tpu as pltpu

EMB = 384
NEXP = 8
HID = 1536
NCLS = 1000
TOPK = 2
_EPS = 1e-5


def _layernorm(v, g, b):
    mu = jnp.mean(v, axis=-1, keepdims=True)
    var = jnp.mean((v - mu) ** 2, axis=-1, keepdims=True)
    return (v - mu) / jnp.sqrt(var + _EPS) * g + b


def _mm_t(a, w):
    return jax.lax.dot_general(
        a, w, (((1,), (1,)), ((), ())), preferred_element_type=jnp.float32
    )


def _mm(a, w):
    return jax.lax.dot_general(
        a, w, (((1,), (0,)), ((), ())), preferred_element_type=jnp.float32
    )


def _gelu(h):
    return 0.5 * h * (1.0 + jax.lax.erf(h * (1.0 / jnp.sqrt(2.0).astype(jnp.float32))))


def _body(cls_ref, pos_ref, rw_ref, rb_ref,
          g1_ref, c1_ref, bv_ref, bo_ref, g2_ref, c2_ref, b1_ref, b2_ref,
          ng_ref, nb_ref, hb_ref,
          hw_hbm, wv_hbm, wo_hbm, w1_hbm, w2_hbm, out_ref,
          iv_ref, is_ref, hw_s, wv_s, wo_s, w1_s, w2_s, sems):
    hw_cp = pltpu.make_async_copy(hw_hbm, hw_s, sems.at[9])
    hw_cp.start()
    tokrow = cls_ref[...] + pos_ref[...]
    logits = _mm_t(tokrow, rw_ref[...]) + rb_ref[...]

    lane = jax.lax.broadcasted_iota(jnp.int32, logits.shape, 1)
    m1 = jnp.max(logits, axis=-1, keepdims=True)
    i1 = jnp.min(jnp.where(logits == m1, lane, NEXP), axis=-1, keepdims=True)
    masked = jnp.where(lane == i1, jnp.full_like(logits, -3.0e38), logits)
    m2 = jnp.max(masked, axis=-1, keepdims=True)
    i2 = jnp.min(jnp.where(masked == m2, lane, NEXP), axis=-1, keepdims=True)

    iv_ref[:, 0:1] = i1
    iv_ref[:, 1:2] = i2
    cp = pltpu.make_async_copy(iv_ref, is_ref, sems.at[8])
    cp.start()

    oh1 = (lane == i1).astype(jnp.float32)
    oh2 = (lane == i2).astype(jnp.float32)

    cp.wait()
    e1 = is_ref[0, 0]
    e2 = is_ref[0, 1]

    cps = []
    for k, e in enumerate((e1, e2)):
        cps.append(pltpu.make_async_copy(wv_hbm.at[e], wv_s.at[k], sems.at[4 * k + 0]))
        cps.append(pltpu.make_async_copy(wo_hbm.at[e], wo_s.at[k], sems.at[4 * k + 1]))
        cps.append(pltpu.make_async_copy(w1_hbm.at[e], w1_s.at[k], sems.at[4 * k + 2]))
        cps.append(pltpu.make_async_copy(w2_hbm.at[e], w2_s.at[k], sems.at[4 * k + 3]))
    for c in cps:
        c.start()

    def pick(oh, ref):
        return _mm(oh, ref[...])

    def expert_out(k, oh):
        xn = _layernorm(tokrow, pick(oh, g1_ref), pick(oh, c1_ref))
        cps[4 * k + 0].wait()
        v = _mm_t(xn, wv_s[k]) + pick(oh, bv_ref)
        cps[4 * k + 1].wait()
        attn = _mm_t(v, wo_s[k]) + pick(oh, bo_ref)
        hmid = tokrow + attn
        hn = _layernorm(hmid, pick(oh, g2_ref), pick(oh, c2_ref))
        cps[4 * k + 2].wait()
        h1 = _gelu(_mm_t(hn, w1_s[k]) + pick(oh, b1_ref))
        cps[4 * k + 3].wait()
        m = _mm_t(h1, w2_s[k]) + pick(oh, b2_ref)
        return hmid + m

    y1 = expert_out(0, oh1)
    y2 = expert_out(1, oh2)
    s = (y1 + y2) * (1.0 / TOPK)
    o = _layernorm(s, ng_ref[...], nb_ref[...])
    hw_cp.wait()
    head = _mm_t(o, hw_s[...]) + hb_ref[...]
    out_ref[...] = jnp.broadcast_to(head, out_ref.shape)


def kernel(x, patch_W, patch_b, cls_token, pos_embed, router_W, router_b,
           ln1_g, ln1_b, Wv, bv, Wo, bo, ln2_g, ln2_b, W1, b1, W2, b2,
           norm_g, norm_b, head_W, head_b):
    Bsz = x.shape[0]
    vmem = pl.BlockSpec(memory_space=pltpu.VMEM)
    hbm = pl.BlockSpec(memory_space=pl.ANY)

    out = pl.pallas_call(
        _body,
        in_specs=[vmem] * 15 + [hbm] * 5,
        out_specs=vmem,
        out_shape=jax.ShapeDtypeStruct((Bsz, NCLS), jnp.float32),
        scratch_shapes=[
            pltpu.VMEM((1, 128), jnp.int32),
            pltpu.SMEM((1, 128), jnp.int32),
            pltpu.VMEM((NCLS, EMB), jnp.float32),
            pltpu.VMEM((TOPK, EMB, EMB), jnp.float32),
            pltpu.VMEM((TOPK, EMB, EMB), jnp.float32),
            pltpu.VMEM((TOPK, HID, EMB), jnp.float32),
            pltpu.VMEM((TOPK, EMB, HID), jnp.float32),
            pltpu.SemaphoreType.DMA((10,)),
        ],
    )(cls_token.reshape(1, EMB), pos_embed[:, 0, :].reshape(1, EMB),
      router_W, router_b.reshape(1, NEXP),
      ln1_g, ln1_b, bv, bo, ln2_g, ln2_b, b1, b2,
      norm_g.reshape(1, EMB), norm_b.reshape(1, EMB),
      head_b.reshape(1, NCLS),
      head_W, Wv, Wo, W1, W2)
    return out


# speedup vs baseline: 2.0631x; 1.0612x over previous
import jax
import jax.numpy as jnp
from jax.experimental import pallas as pl
from jax.experimental.pallas import tpu as pltpu

EMB = 384
NEXP = 8
HID = 1536
NCLS = 1000
TOPK = 2
_EPS = 1e-5


def _layernorm(v, g, b):
    mu = jnp.mean(v, axis=-1, keepdims=True)
    var = jnp.mean((v - mu) ** 2, axis=-1, keepdims=True)
    return (v - mu) / jnp.sqrt(var + _EPS) * g + b


def _mm_t(a, w):
    return jax.lax.dot_general(
        a, w, (((1,), (1,)), ((), ())), preferred_element_type=jnp.float32
    )


def _mm(a, w):
    return jax.lax.dot_general(
        a, w, (((1,), (0,)), ((), ())), preferred_element_type=jnp.float32
    )


def _gelu(h):
    return 0.5 * h * (1.0 + jax.lax.erf(h * (1.0 / jnp.sqrt(2.0).astype(jnp.float32))))


def _body(cls_ref, pos_ref, rw_ref, rb_ref,
          g1_ref, c1_ref, bv_ref, bo_ref, g2_ref, c2_ref, b1_ref, b2_ref,
          ng_ref, nb_ref, hb_ref,
          hw_hbm, wv_hbm, wo_hbm, w1_hbm, w2_hbm, out_ref,
          iv_ref, is_ref, hw_s, wv_s, wo_s, w1_s, w2_s, sems):
    hw_cp = pltpu.make_async_copy(hw_hbm, hw_s, sems.at[9])
    hw_cp.start()
    tokrow = cls_ref[...] + pos_ref[...]
    logits = _mm_t(tokrow, rw_ref[...]) + rb_ref[...]

    lane = jax.lax.broadcasted_iota(jnp.int32, logits.shape, 1)
    m1 = jnp.max(logits, axis=-1, keepdims=True)
    i1 = jnp.min(jnp.where(logits == m1, lane, NEXP), axis=-1, keepdims=True)
    masked = jnp.where(lane == i1, jnp.full_like(logits, -3.0e38), logits)
    m2 = jnp.max(masked, axis=-1, keepdims=True)
    i2 = jnp.min(jnp.where(masked == m2, lane, NEXP), axis=-1, keepdims=True)

    iv_ref[:, 0:1] = i1
    iv_ref[:, 1:2] = i2
    cp = pltpu.make_async_copy(iv_ref, is_ref, sems.at[8])
    cp.start()

    oh1 = (lane == i1).astype(jnp.float32)
    oh2 = (lane == i2).astype(jnp.float32)

    def pick(oh, ref):
        return _mm(oh, ref[...])

    xn0 = _layernorm(tokrow, pick(oh1, g1_ref), pick(oh1, c1_ref))
    xn1 = _layernorm(tokrow, pick(oh2, g1_ref), pick(oh2, c1_ref))
    bvs = (pick(oh1, bv_ref), pick(oh2, bv_ref))
    bos = (pick(oh1, bo_ref), pick(oh2, bo_ref))
    g2s = (pick(oh1, g2_ref), pick(oh2, g2_ref))
    c2s = (pick(oh1, c2_ref), pick(oh2, c2_ref))
    b1s = (pick(oh1, b1_ref), pick(oh2, b1_ref))
    b2s = (pick(oh1, b2_ref), pick(oh2, b2_ref))

    cp.wait()
    e1 = is_ref[0, 0]
    e2 = is_ref[0, 1]

    cps = []
    for k, e in enumerate((e1, e2)):
        cps.append(pltpu.make_async_copy(wv_hbm.at[e], wv_s.at[k], sems.at[4 * k + 0]))
        cps.append(pltpu.make_async_copy(wo_hbm.at[e], wo_s.at[k], sems.at[4 * k + 1]))
        cps.append(pltpu.make_async_copy(w1_hbm.at[e], w1_s.at[k], sems.at[4 * k + 2]))
        cps.append(pltpu.make_async_copy(w2_hbm.at[e], w2_s.at[k], sems.at[4 * k + 3]))
    for c in cps:
        c.start()

    def expert_out(k, xn):
        cps[4 * k + 0].wait()
        v = _mm_t(xn, wv_s[k]) + bvs[k]
        cps[4 * k + 1].wait()
        attn = _mm_t(v, wo_s[k]) + bos[k]
        hmid = tokrow + attn
        hn = _layernorm(hmid, g2s[k], c2s[k])
        cps[4 * k + 2].wait()
        h1 = _gelu(_mm_t(hn, w1_s[k]) + b1s[k])
        cps[4 * k + 3].wait()
        m = _mm_t(h1, w2_s[k]) + b2s[k]
        return hmid + m

    y1 = expert_out(0, xn0)
    y2 = expert_out(1, xn1)
    s = (y1 + y2) * (1.0 / TOPK)
    o = _layernorm(s, ng_ref[...], nb_ref[...])
    hw_cp.wait()
    head = _mm_t(o, hw_s[...]) + hb_ref[...]
    out_ref[...] = jnp.broadcast_to(head, out_ref.shape)


def kernel(x, patch_W, patch_b, cls_token, pos_embed, router_W, router_b,
           ln1_g, ln1_b, Wv, bv, Wo, bo, ln2_g, ln2_b, W1, b1, W2, b2,
           norm_g, norm_b, head_W, head_b):
    Bsz = x.shape[0]
    vmem = pl.BlockSpec(memory_space=pltpu.VMEM)
    hbm = pl.BlockSpec(memory_space=pl.ANY)

    out = pl.pallas_call(
        _body,
        in_specs=[vmem] * 15 + [hbm] * 5,
        out_specs=vmem,
        out_shape=jax.ShapeDtypeStruct((Bsz, NCLS), jnp.float32),
        scratch_shapes=[
            pltpu.VMEM((1, 128), jnp.int32),
            pltpu.SMEM((1, 128), jnp.int32),
            pltpu.VMEM((NCLS, EMB), jnp.float32),
            pltpu.VMEM((TOPK, EMB, EMB), jnp.float32),
            pltpu.VMEM((TOPK, EMB, EMB), jnp.float32),
            pltpu.VMEM((TOPK, HID, EMB), jnp.float32),
            pltpu.VMEM((TOPK, EMB, HID), jnp.float32),
            pltpu.SemaphoreType.DMA((10,)),
        ],
    )(cls_token.reshape(1, EMB), pos_embed[:, 0, :].reshape(1, EMB),
      router_W, router_b.reshape(1, NEXP),
      ln1_g, ln1_b, bv, bo, ln2_g, ln2_b, b1, b2,
      norm_g.reshape(1, EMB), norm_b.reshape(1, EMB),
      head_b.reshape(1, NCLS),
      head_W, Wv, Wo, W1, W2)
    return out
